# baseline (speedup 1.0000x reference)
import jax
import jax.numpy as jnp
from jax.experimental import pallas as pl

_BLK = 128


def _diag_block(fill_ref, x_ref, o_ref):
    rows = jax.lax.broadcasted_iota(jnp.int32, x_ref.shape, 0)
    cols = jax.lax.broadcasted_iota(jnp.int32, x_ref.shape, 1)
    o_ref[...] = jnp.where(rows == cols, fill_ref[0, 0], x_ref[...])


def kernel(x, fill_value):
    n = min(x.shape)
    fill = jnp.asarray(fill_value, x.dtype).reshape(1, 1)
    return pl.pallas_call(
        _diag_block,
        grid=(n // _BLK,),
        in_specs=[
            pl.BlockSpec((1, 1), lambda i: (0, 0)),
            pl.BlockSpec((_BLK, _BLK), lambda i: (i, i)),
        ],
        out_specs=pl.BlockSpec((_BLK, _BLK), lambda i: (i, i)),
        out_shape=jax.ShapeDtypeStruct(x.shape, x.dtype),
        input_output_aliases={1: 0},
    )(fill, x)
